# Initial kernel scaffold; baseline (speedup 1.0000x reference)
import jax
import jax.numpy as jnp
from jax.experimental import pallas as pl


def kernel(z_from_encoder, codebook, codebook_weight, flg_train):
    raise NotImplementedError("write your pallas kernel here")


# speedup vs baseline: 1.0197x; 1.0197x over previous
import functools

import jax
import jax.numpy as jnp
from jax.experimental import pallas as pl
from jax.experimental.pallas import tpu as pltpu

B, C, K, HW = 16, 256, 1024, 1024
N = B * HW


def _vq_body(z_ref, cb_ref, zq_ref, loss_ref, perp_ref, counts_ref, sse_ref):
    b = pl.program_id(0)
    zb = z_ref[0]
    cb = cb_ref[...]
    c2 = jnp.sum(cb * cb, axis=1, keepdims=True)
    z2 = jnp.sum(zb * zb, axis=0, keepdims=True)
    m = jax.lax.dot_general(cb, zb, (((1,), (0,)), ((), ())),
                            preferred_element_type=jnp.float32)
    scores = (z2 + c2) - 2.0 * m
    idx = jnp.argmin(scores, axis=0)
    onehot = (jax.lax.broadcasted_iota(jnp.int32, (K, HW), 0)
              == idx[None, :]).astype(jnp.float32)
    zq = jax.lax.dot_general(cb, onehot, (((0,), (0,)), ((), ())),
                             preferred_element_type=jnp.float32)
    zq_ref[0] = zb + (zq - zb)

    part_counts = jnp.sum(onehot, axis=1, keepdims=True)
    part_sse = jnp.sum((zq - zb) ** 2)[None, None]

    @pl.when(b == 0)
    def _init():
        counts_ref[...] = part_counts
        sse_ref[...] = part_sse

    @pl.when(b > 0)
    def _acc():
        counts_ref[...] += part_counts
        sse_ref[...] += part_sse

    @pl.when(b == pl.num_programs(0) - 1)
    def _fin():
        loss_ref[...] = 1.25 / (B * C * HW) * sse_ref[...]
        e_mean = counts_ref[...] * (1.0 / N)
        ent = jnp.sum(e_mean * jnp.log(e_mean + 1e-10))
        perp_ref[...] = jnp.exp(-ent)[None, None]


def _vq_call(z, cb, interpret=False):
    return pl.pallas_call(
        _vq_body,
        grid=(B,),
        in_specs=[
            pl.BlockSpec((1, C, HW), lambda b: (b, 0, 0)),
            pl.BlockSpec((K, C), lambda b: (0, 0)),
        ],
        out_specs=[
            pl.BlockSpec((1, C, HW), lambda b: (b, 0, 0)),
            pl.BlockSpec((1, 1), lambda b: (0, 0)),
            pl.BlockSpec((1, 1), lambda b: (0, 0)),
        ],
        out_shape=[
            jax.ShapeDtypeStruct((B, C, HW), jnp.float32),
            jax.ShapeDtypeStruct((1, 1), jnp.float32),
            jax.ShapeDtypeStruct((1, 1), jnp.float32),
        ],
        scratch_shapes=[
            pltpu.VMEM((K, 1), jnp.float32),
            pltpu.VMEM((1, 1), jnp.float32),
        ],
        interpret=interpret,
    )(z, cb)


def kernel(z_from_encoder, codebook, codebook_weight, flg_train):
    z = z_from_encoder.reshape(B, C, HW)
    zq, loss, perp = _vq_call(z, codebook)
    loss = jnp.where(flg_train != 0, loss[0, 0], jnp.float32(0.0))
    return (zq.reshape(B, C, 32, 32), loss, perp[0, 0])


# speedup vs baseline: 1.1122x; 1.0907x over previous
import functools

import jax
import jax.numpy as jnp
from jax.experimental import pallas as pl
from jax.experimental.pallas import tpu as pltpu

B, C, K, HW = 16, 256, 1024, 1024
N = B * HW


def _vq_body(z_ref, cb_ref, zq_ref, loss_ref, perp_ref, counts_ref, sse_ref):
    b = pl.program_id(0)
    zb = z_ref[0]
    cb = cb_ref[...]
    c2 = jnp.sum(cb * cb, axis=1, keepdims=True)
    z2 = jnp.sum(zb * zb, axis=0, keepdims=True)
    m = jax.lax.dot_general(cb, zb, (((1,), (0,)), ((), ())),
                            preferred_element_type=jnp.float32)
    scores = (z2 + c2) - 2.0 * m
    min_s = jnp.min(scores, axis=0, keepdims=True)
    onehot = (scores == min_s).astype(jnp.bfloat16)
    cb_bf = cb.astype(jnp.bfloat16)
    zq = jax.lax.dot_general(cb_bf, onehot, (((0,), (0,)), ((), ())),
                             preferred_element_type=jnp.float32)
    zq_ref[0] = zq

    ones8 = jnp.ones((8, HW), jnp.bfloat16)
    part_counts = jax.lax.dot_general(ones8, onehot, (((1,), (1,)), ((), ())),
                                      preferred_element_type=jnp.float32)
    part_sse = jnp.sum((zq - zb) ** 2)[None, None]

    @pl.when(b == 0)
    def _init():
        counts_ref[...] = part_counts
        sse_ref[...] = part_sse

    @pl.when(b > 0)
    def _acc():
        counts_ref[...] += part_counts
        sse_ref[...] += part_sse

    @pl.when(b == pl.num_programs(0) - 1)
    def _fin():
        loss_ref[...] = 1.25 / (B * C * HW) * sse_ref[...]
        e_mean = counts_ref[0:1, :] * (1.0 / N)
        ent = jnp.sum(e_mean * jnp.log(e_mean + 1e-10))
        perp_ref[...] = jnp.exp(-ent)[None, None]


def _vq_call(z, cb, interpret=False):
    return pl.pallas_call(
        _vq_body,
        grid=(B,),
        in_specs=[
            pl.BlockSpec((1, C, HW), lambda b: (b, 0, 0)),
            pl.BlockSpec((K, C), lambda b: (0, 0)),
        ],
        out_specs=[
            pl.BlockSpec((1, C, HW), lambda b: (b, 0, 0)),
            pl.BlockSpec((1, 1), lambda b: (0, 0)),
            pl.BlockSpec((1, 1), lambda b: (0, 0)),
        ],
        out_shape=[
            jax.ShapeDtypeStruct((B, C, HW), jnp.float32),
            jax.ShapeDtypeStruct((1, 1), jnp.float32),
            jax.ShapeDtypeStruct((1, 1), jnp.float32),
        ],
        scratch_shapes=[
            pltpu.VMEM((8, K), jnp.float32),
            pltpu.VMEM((1, 1), jnp.float32),
        ],
        interpret=interpret,
    )(z, cb)


def kernel(z_from_encoder, codebook, codebook_weight, flg_train):
    z = z_from_encoder.reshape(B, C, HW)
    zq, loss, perp = _vq_call(z, codebook)
    loss = jnp.where(flg_train != 0, loss[0, 0], jnp.float32(0.0))
    return (zq.reshape(B, C, 32, 32), loss, perp[0, 0])
